# baseline (speedup 1.0000x reference)
import functools

import jax
import jax.numpy as jnp
from jax import lax
from jax.experimental import pallas as pl
from jax.experimental.pallas import tpu as pltpu
from jax.experimental.pallas import tpu_sc as plsc

NC, NS, L = 2, 16, 16


def _permute(x, idx):
    dnums = lax.GatherDimensionNumbers(
        offset_dims=(), collapsed_slice_dims=(0,), start_index_map=(0,))
    return lax.gather(
        x, idx[:, None], dnums, (1,),
        mode=lax.GatherScatterMode.PROMISE_IN_BOUNDS)
NW = NC * NS
IDXW = 128


def _sc_scores(c_idx, pn_idx_rows, in_emb, out_emb, *, B, R, D, CB, RP):
    b_per_w = B // NW
    chunks = b_per_w // CB
    idx_rows = CB * R // IDXW
    rows_chunk = CB * R

    mesh = plsc.VectorSubcoreMesh(
        core_axis_name="c", subcore_axis_name="s", num_cores=NC, num_subcores=NS
    )

    @functools.partial(
        pl.kernel,
        out_type=jax.ShapeDtypeStruct((B, RP), jnp.float32),
        mesh=mesh,
        scratch_types=[
            pltpu.VMEM((CB,), jnp.int32),
            pltpu.VMEM((rows_chunk,), jnp.int32),
            pltpu.VMEM((CB, D), jnp.float32),
            pltpu.VMEM((rows_chunk, D), jnp.float32),
            pltpu.VMEM((CB, RP), jnp.float32),
            pltpu.SemaphoreType.DMA,
        ],
        compiler_params=pltpu.CompilerParams(use_tc_tiling_on_sc=False),
    )
    def k(cidx_hbm, pnidx_hbm, in_hbm, out_hbm, scores_hbm,
          cidx_v, pnidx_v, crows_v, rows_v, scores_v, sem):
        wid = lax.axis_index("s") * NC + lax.axis_index("c")

        def chunk_body(t, _):
            b0 = wid * b_per_w + t * CB
            pltpu.sync_copy(cidx_hbm.at[pl.ds(b0, CB)], cidx_v)
            pltpu.sync_copy(
                pnidx_hbm.at[pl.ds(b0 * R, rows_chunk)], pnidx_v)
            cps = [pltpu.async_copy(in_hbm.at[cidx_v], crows_v, sem)]
            for i in range(idx_rows):
                cps.append(pltpu.async_copy(
                    out_hbm.at[pnidx_v.at[pl.ds(i * IDXW, IDXW)]],
                    rows_v.at[pl.ds(i * IDXW, IDXW)], sem))
            for cp in cps:
                cp.wait()

            lane = lax.broadcasted_iota(jnp.int32, (L,), 0)
            masks = [(lane & (1 << k)) != 0 for k in range(4)]
            perms = [lane ^ (1 << k) for k in range(4)]

            def dot_group(row0, cvecs, nrows):
                vecs = []
                for jj in range(L):
                    if jj < nrows:
                        r = row0 + jj
                        acc = rows_v[r, pl.ds(0, L)] * cvecs[0]
                        for q in range(1, D // L):
                            acc += rows_v[r, pl.ds(q * L, L)] * cvecs[q]
                        vecs.append(acc)
                    else:
                        vecs.append(jnp.zeros((L,), jnp.float32))
                for k in (3, 2, 1, 0):
                    half = len(vecs) // 2
                    nxt = []
                    for i in range(half):
                        a, b = vecs[i], vecs[i + half]
                        sel = jnp.where(masks[k], b, a)
                        sel2 = jnp.where(masks[k], a, b)
                        nxt.append(sel + _permute(sel2, perms[k]))
                    vecs = nxt
                return vecs[0]

            def b_body(b, _):
                cvecs = [crows_v[b, pl.ds(q * L, L)] for q in range(D // L)]
                for g in range(RP // L):
                    glen = max(0, min(L, R - g * L))
                    scores_v[b, pl.ds(g * L, L)] = dot_group(
                        b * R + g * L, cvecs, glen)
                return 0

            lax.fori_loop(0, CB, b_body, 0)
            pltpu.sync_copy(scores_v, scores_hbm.at[pl.ds(b0, CB)])
            return 0

        lax.fori_loop(0, chunks, chunk_body, 0)

    return k(c_idx, pn_idx_rows, in_emb, out_emb)


def _tc_loss_body(scores_ref, out_ref, *, P, R, B):
    s = scores_ref[...]
    col = lax.broadcasted_iota(jnp.int32, s.shape, 1)
    is_p = col < P
    x = jnp.where(is_p, s, -s)
    ls = jnp.minimum(x, 0.0) - jnp.log1p(jnp.exp(-jnp.abs(x)))
    w = jnp.where(is_p, 1.0 / (B * P), jnp.where(col < R, 1.0 / B, 0.0))
    out_ref[0, 0] = -jnp.sum(ls * w)


def kernel(c_word, p_word, n_word, in_emb, out_emb):
    B, P = p_word.shape
    N = n_word.shape[1]
    D = in_emb.shape[1]
    R = P + N

    c_idx = c_word.reshape(B).astype(jnp.int32)
    pn_idx = jnp.concatenate(
        [p_word.astype(jnp.int32), n_word.astype(jnp.int32)], axis=1)
    pn_idx_rows = pn_idx.reshape(B * R)

    RP = (R + L - 1) // L * L
    scores = _sc_scores(c_idx, pn_idx_rows, in_emb, out_emb,
                        B=B, R=R, D=D, CB=16, RP=RP)

    loss = pl.pallas_call(
        functools.partial(_tc_loss_body, P=P, R=R, B=B),
        out_shape=jax.ShapeDtypeStruct((1, 1), jnp.float32),
        out_specs=pl.BlockSpec(memory_space=pltpu.SMEM),
    )(scores)
    return loss[0, 0]


# speedup vs baseline: 1.0529x; 1.0529x over previous
import functools

import jax
import jax.numpy as jnp
from jax import lax
from jax.experimental import pallas as pl
from jax.experimental.pallas import tpu as pltpu
from jax.experimental.pallas import tpu_sc as plsc

NC, NS, L = 2, 16, 16
NW = NC * NS
IDXW = 128


def _permute(x, idx):
    dnums = lax.GatherDimensionNumbers(
        offset_dims=(), collapsed_slice_dims=(0,), start_index_map=(0,))
    return lax.gather(
        x, idx[:, None], dnums, (1,),
        mode=lax.GatherScatterMode.PROMISE_IN_BOUNDS)


def _sc_scores(c_idx, pn_idx_flat, in_emb, out_emb, *, B, R, D, CB, RP):
    b_per_w = B // NW
    chunks = b_per_w // CB
    idx_rows = CB * R // IDXW
    rows_chunk = CB * R
    assert chunks % 2 == 0 and chunks >= 4

    mesh = plsc.VectorSubcoreMesh(
        core_axis_name="c", subcore_axis_name="s", num_cores=NC, num_subcores=NS
    )

    @functools.partial(
        pl.kernel,
        out_type=jax.ShapeDtypeStruct((B, RP), jnp.float32),
        mesh=mesh,
        scratch_types=[
            pltpu.VMEM((2, CB), jnp.int32),
            pltpu.VMEM((2, rows_chunk), jnp.int32),
            pltpu.VMEM((2, CB, D), jnp.float32),
            pltpu.VMEM((2, rows_chunk, D), jnp.float32),
            pltpu.VMEM((CB, RP), jnp.float32),
            pltpu.SemaphoreType.DMA,
            pltpu.SemaphoreType.DMA,
        ],
        compiler_params=pltpu.CompilerParams(use_tc_tiling_on_sc=False),
    )
    def k(cidx_hbm, pnidx_hbm, in_hbm, out_hbm, scores_hbm,
          cidx_v, pnidx_v, crows_v, rows_v, scores_v, sem0, sem1):
        wid = lax.axis_index("s") * NC + lax.axis_index("c")
        sems = (sem0, sem1)

        lane = lax.broadcasted_iota(jnp.int32, (L,), 0)
        masks = [(lane & (1 << k)) != 0 for k in range(4)]
        perms = [lane ^ (1 << k) for k in range(4)]

        def fire(t, s):
            b0 = wid * b_per_w + t * CB
            pltpu.sync_copy(cidx_hbm.at[pl.ds(b0, CB)], cidx_v.at[s])
            pltpu.sync_copy(pnidx_hbm.at[pl.ds(b0 * R, rows_chunk)],
                            pnidx_v.at[s])
            pltpu.async_copy(in_hbm.at[cidx_v.at[s]], crows_v.at[s], sems[s])
            for i in range(idx_rows):
                pltpu.async_copy(
                    out_hbm.at[pnidx_v.at[s, pl.ds(i * IDXW, IDXW)]],
                    rows_v.at[s, pl.ds(i * IDXW, IDXW)], sems[s])

        def drain(s):
            pltpu.make_async_copy(
                in_hbm.at[cidx_v.at[s]], crows_v.at[s], sems[s]).wait()
            for i in range(idx_rows):
                pltpu.make_async_copy(
                    out_hbm.at[pnidx_v.at[s, pl.ds(i * IDXW, IDXW)]],
                    rows_v.at[s, pl.ds(i * IDXW, IDXW)], sems[s]).wait()

        def dot_group(s, row0, cvecs, nrows):
            vecs = []
            for jj in range(L):
                if jj < nrows:
                    r = row0 + jj
                    acc = rows_v[s, r, pl.ds(0, L)] * cvecs[0]
                    for q in range(1, D // L):
                        acc += rows_v[s, r, pl.ds(q * L, L)] * cvecs[q]
                    vecs.append(acc)
                else:
                    vecs.append(jnp.zeros((L,), jnp.float32))
            for k in (3, 2, 1, 0):
                half = len(vecs) // 2
                nxt = []
                for i in range(half):
                    a, b = vecs[i], vecs[i + half]
                    sel = jnp.where(masks[k], b, a)
                    sel2 = jnp.where(masks[k], a, b)
                    nxt.append(sel + _permute(sel2, perms[k]))
                vecs = nxt
            return vecs[0]

        def compute(t, s):
            b0 = wid * b_per_w + t * CB

            def b_body(b, _):
                cvecs = [crows_v[s, b, pl.ds(q * L, L)] for q in range(D // L)]
                for g in range(RP // L):
                    glen = max(0, min(L, R - g * L))
                    scores_v[b, pl.ds(g * L, L)] = dot_group(
                        s, b * R + g * L, cvecs, glen)
                return 0

            lax.fori_loop(0, CB, b_body, 0)
            pltpu.sync_copy(scores_v, scores_hbm.at[pl.ds(b0, CB)])

        fire(0, 0)

        def body(t2, _):
            t = 2 * t2
            fire(t + 1, 1)
            drain(0)
            compute(t, 0)
            fire(t + 2, 0)
            drain(1)
            compute(t + 1, 1)
            return 0

        lax.fori_loop(0, chunks // 2 - 1, body, 0)
        t = chunks - 2
        fire(t + 1, 1)
        drain(0)
        compute(t, 0)
        drain(1)
        compute(t + 1, 1)

    return k(c_idx, pn_idx_flat, in_emb, out_emb)


def _tc_loss_body(scores_ref, out_ref, *, P, R, B):
    s = scores_ref[...]
    col = lax.broadcasted_iota(jnp.int32, s.shape, 1)
    is_p = col < P
    x = jnp.where(is_p, s, -s)
    ls = jnp.minimum(x, 0.0) - jnp.log1p(jnp.exp(-jnp.abs(x)))
    w = jnp.where(is_p, 1.0 / (B * P), jnp.where(col < R, 1.0 / B, 0.0))
    out_ref[0, 0] = -jnp.sum(ls * w)


def kernel(c_word, p_word, n_word, in_emb, out_emb):
    B, P = p_word.shape
    N = n_word.shape[1]
    D = in_emb.shape[1]
    R = P + N

    c_idx = c_word.reshape(B).astype(jnp.int32)
    pn_idx = jnp.concatenate(
        [p_word.astype(jnp.int32), n_word.astype(jnp.int32)], axis=1)
    pn_idx_flat = pn_idx.reshape(B * R)

    RP = (R + L - 1) // L * L
    scores = _sc_scores(c_idx, pn_idx_flat, in_emb, out_emb,
                        B=B, R=R, D=D, CB=16, RP=RP)

    loss = pl.pallas_call(
        functools.partial(_tc_loss_body, P=P, R=R, B=B),
        out_shape=jax.ShapeDtypeStruct((1, 1), jnp.float32),
        out_specs=pl.BlockSpec(memory_space=pltpu.SMEM),
    )(scores)
    return loss[0, 0]
